# baseline (speedup 1.0000x reference)
import functools
import math

import jax
import jax.numpy as jnp
from jax import lax
from jax.experimental import pallas as pl
from jax.experimental.pallas import tpu as pltpu
from jax.experimental.pallas import tpu_sc as plsc

BATCH = 4096
SEQ = 50
D = 128
NC = 2
NS = 16
NW = NC * NS
COLS_PER_W = BATCH // NW
SCALE = math.sqrt(128.0)
L = 16
NBUF = 5


def _emb_body(table_hbm, idx_hbm, out_hbm, idx_v, big, gsems, ssems):
    c = lax.axis_index("c")
    s = lax.axis_index("s")
    wid = s * NC + c
    col0 = wid * COLS_PER_W
    pltpu.sync_copy(idx_hbm.at[:, pl.ds(col0, COLS_PER_W)], idx_v)

    for b in range(NBUF - 2):
        pltpu.async_copy(
            table_hbm.at[idx_v.at[b]],
            big.at[pl.ds(b * COLS_PER_W, COLS_PER_W)], gsems.at[b])

    @pl.loop(0, SEQ)
    def _step(t):
        bsel = lax.rem(t, NBUF)
        nbsel = lax.rem(t + NBUF - 2, NBUF)
        bslc = big.at[pl.ds(bsel * COLS_PER_W, COLS_PER_W)]
        nbslc = big.at[pl.ds(nbsel * COLS_PER_W, COLS_PER_W)]

        @pl.when(t >= 2)
        def _drain():
            pltpu.make_async_copy(
                nbslc, out_hbm.at[0, pl.ds(col0, COLS_PER_W)],
                ssems.at[nbsel]).wait()

        @pl.when(t + NBUF - 2 < SEQ)
        def _lookahead():
            pltpu.async_copy(
                table_hbm.at[idx_v.at[t + NBUF - 2]], nbslc, gsems.at[nbsel])

        pltpu.make_async_copy(
            table_hbm.at[idx_v.at[t]], bslc, gsems.at[bsel]).wait()

        row0 = bsel * COLS_PER_W

        @pl.loop(0, COLS_PER_W)
        def _row(r):
            for seg in range(D // L):
                sl = pl.ds(seg * L, L)
                big[row0 + r, sl] = big[row0 + r, sl] * SCALE

        pltpu.async_copy(
            bslc, out_hbm.at[t, pl.ds(col0, COLS_PER_W)], ssems.at[bsel])

    for t in (SEQ - 2, SEQ - 1):
        pltpu.make_async_copy(
            big.at[pl.ds((t % NBUF) * COLS_PER_W, COLS_PER_W)],
            out_hbm.at[0, pl.ds(col0, COLS_PER_W)],
            ssems.at[t % NBUF]).wait()


@functools.partial(jax.jit, static_argnames=())
def _emb_call(idx_t, table):
    mesh = plsc.VectorSubcoreMesh(core_axis_name="c", subcore_axis_name="s")
    k = pl.kernel(
        _emb_body,
        out_type=jax.ShapeDtypeStruct((SEQ, BATCH, D), jnp.float32),
        mesh=mesh,
        scratch_types=(
            [pltpu.VMEM((SEQ, COLS_PER_W), jnp.int32),
             pltpu.VMEM((NBUF * COLS_PER_W, D), jnp.float32),
             pltpu.SemaphoreType.DMA((NBUF,)),
             pltpu.SemaphoreType.DMA((NBUF,))]
        ),
    )
    return k(table, idx_t)


def kernel(x, table):
    idx_t = x.astype(jnp.int32).T
    out_t = _emb_call(idx_t, table)
    return out_t.transpose(1, 0, 2)
